# Initial kernel scaffold; baseline (speedup 1.0000x reference)
import jax
import jax.numpy as jnp
from jax.experimental import pallas as pl


def kernel(x, type_emb, index):
    raise NotImplementedError("write your pallas kernel here")


# speedup vs baseline: 1.8389x; 1.8389x over previous
import jax
import jax.numpy as jnp
from jax.experimental import pallas as pl
from jax.experimental.pallas import tpu as pltpu

_ROW_BLK = 512


def _cat_kernel(idx_ref, x_ref, temb_ref, out_ref):
    d_in = x_ref.shape[-1]
    d_emb = temb_ref.shape[-1]
    out_ref[:, :d_in] = x_ref[...]
    idx = idx_ref[0]
    row = temb_ref[pl.ds(idx, 1), :]
    out_ref[:, d_in:] = jnp.broadcast_to(row, (out_ref.shape[0], d_emb))


def kernel(x, type_emb, index):
    b, s, d = x.shape
    n = b * s
    d_emb = type_emb.shape[-1]
    x2 = x.reshape(n, d)
    idx = jnp.asarray(index, jnp.int32).reshape((1,))
    out = pl.pallas_call(
        _cat_kernel,
        grid_spec=pltpu.PrefetchScalarGridSpec(
            num_scalar_prefetch=1,
            grid=(n // _ROW_BLK,),
            in_specs=[
                pl.BlockSpec((_ROW_BLK, d), lambda i, s_ref: (i, 0)),
                pl.BlockSpec(type_emb.shape, lambda i, s_ref: (0, 0)),
            ],
            out_specs=pl.BlockSpec((_ROW_BLK, d + d_emb), lambda i, s_ref: (i, 0)),
        ),
        out_shape=jax.ShapeDtypeStruct((n, d + d_emb), x.dtype),
    )(idx, x2, type_emb)
    return out.reshape(b, s, d + d_emb)


# speedup vs baseline: 1.8656x; 1.0145x over previous
import jax
import jax.numpy as jnp
from jax.experimental import pallas as pl
from jax.experimental.pallas import tpu as pltpu

_ROW_BLK = 1024


def _cat_kernel(idx_ref, x_ref, temb_ref, out_ref):
    d_in = x_ref.shape[-1]
    d_emb = temb_ref.shape[-1]
    out_ref[:, :d_in] = x_ref[...]
    idx = idx_ref[0]
    row = temb_ref[pl.ds(idx, 1), :]
    out_ref[:, d_in:] = jnp.broadcast_to(row, (out_ref.shape[0], d_emb))


def kernel(x, type_emb, index):
    b, s, d = x.shape
    n = b * s
    d_emb = type_emb.shape[-1]
    x2 = x.reshape(n, d)
    idx = jnp.asarray(index, jnp.int32).reshape((1,))
    out = pl.pallas_call(
        _cat_kernel,
        grid_spec=pltpu.PrefetchScalarGridSpec(
            num_scalar_prefetch=1,
            grid=(n // _ROW_BLK,),
            in_specs=[
                pl.BlockSpec((_ROW_BLK, d), lambda i, s_ref: (i, 0)),
                pl.BlockSpec(type_emb.shape, lambda i, s_ref: (0, 0)),
            ],
            out_specs=pl.BlockSpec((_ROW_BLK, d + d_emb), lambda i, s_ref: (i, 0)),
        ),
        out_shape=jax.ShapeDtypeStruct((n, d + d_emb), x.dtype),
    )(idx, x2, type_emb)
    return out.reshape(b, s, d + d_emb)
